# Initial kernel scaffold; baseline (speedup 1.0000x reference)
import jax
import jax.numpy as jnp
from jax.experimental import pallas as pl


def kernel(users, businesses, user_user_neigh, user_business_neigh, user_city_neigh, user_category_neigh, business_business_neigh, business_user_neigh, business_city_neigh, business_category_neigh, W_user, b_user, W_business, b_business, W_city, b_city, W_category, b_category):
    raise NotImplementedError("write your pallas kernel here")


# speedup vs baseline: 2.2276x; 2.2276x over previous
import jax
import jax.numpy as jnp
from jax.experimental import pallas as pl
from jax.experimental.pallas import tpu as pltpu

F = 4
D = 32
FD = F * D
B = 512
K = 8
NITER = 3
BK = B * K
NBIG = 4096
NCITY = 128
NCAT = 512

BB = 64
KBLK = 512
NB = B // BB
NK = NBIG // KBLK
ISQ = 1.0 / (D ** 0.5)


def _facet_mats():
    d_of = jax.lax.broadcasted_iota(jnp.int32, (FD, F), 0) // D
    f_of = jax.lax.broadcasted_iota(jnp.int32, (FD, F), 1)
    S = (d_of == f_of).astype(jnp.float32)
    d_of_t = jax.lax.broadcasted_iota(jnp.int32, (F, FD), 1) // D
    f_of_t = jax.lax.broadcasted_iota(jnp.int32, (F, FD), 0)
    St = (d_of_t == f_of_t).astype(jnp.float32)
    return S, St


def _softmax_groups(scores, rows, S_t):
    s = scores.reshape(BB, rows, F)
    m = jnp.max(s, axis=1, keepdims=True)
    e = jnp.exp(s - m)
    a = e / jnp.sum(e, axis=1, keepdims=True)
    return jnp.dot(a.reshape(BB * rows, F), S_t,
                   preferred_element_type=jnp.float32)


def _homo(t, n, S, St):
    trep = jnp.broadcast_to(t[:, None, :], (BB, K, FD)).reshape(BB * K, FD)
    scores = jnp.dot(n * trep, S, preferred_element_type=jnp.float32) * ISQ
    ab = _softmax_groups(scores, K, St)
    return jnp.sum((ab * n).reshape(BB, K, FD), axis=1)


def _hete(t, h, S, St):
    u = t
    for _ in range(NITER):
        urep = jnp.broadcast_to(u[:, None, :], (BB, F, FD)).reshape(BB * F, FD)
        scores = jnp.dot(h * urep, S, preferred_element_type=jnp.float32) * ISQ
        ab = _softmax_groups(scores, F, St)
        agg = jnp.sum((ab * h).reshape(BB, F, FD), axis=1)
        u = t + agg
        sq = jnp.dot(u * u, S, preferred_element_type=jnp.float32)
        denom = jnp.dot(jnp.sqrt(sq), St,
                        preferred_element_type=jnp.float32) + 1e-8
        u = u / denom
    return u


def _body(users_r, bus_r, uu_r, ub_r, uc_r, ucat_r, bb_r, bu_r, bc_r, bcat_r,
          wu_r, bu_b_r, wb_r, bb_b_r, wc_r, bc_b_r, wcat_r, bcat_b_r,
          out_r,
          eu_s, eb_s, nuu_s, nub_s, nuc_s, nucat_s, nbb_s, nbu_s, nbc_s,
          nbcat_s):
    k = pl.program_id(1)

    @pl.when(k == 0)
    def _init():
        eu_s[...] = jnp.broadcast_to(bu_b_r[...], (BB, FD))
        eb_s[...] = jnp.broadcast_to(bb_b_r[...], (BB, FD))
        nuu_s[...] = jnp.broadcast_to(bu_b_r[...], (BK // NB, FD))
        nub_s[...] = jnp.broadcast_to(bb_b_r[...], (BK // NB, FD))
        nbb_s[...] = jnp.broadcast_to(bb_b_r[...], (BK // NB, FD))
        nbu_s[...] = jnp.broadcast_to(bu_b_r[...], (BK // NB, FD))
        nuc_s[...] = jnp.dot(uc_r[...], wc_r[...],
                             preferred_element_type=jnp.float32) + bc_b_r[...]
        nbc_s[...] = jnp.dot(bc_r[...], wc_r[...],
                             preferred_element_type=jnp.float32) + bc_b_r[...]
        nucat_s[...] = jnp.dot(ucat_r[...], wcat_r[...],
                               preferred_element_type=jnp.float32) + bcat_b_r[...]
        nbcat_s[...] = jnp.dot(bcat_r[...], wcat_r[...],
                               preferred_element_type=jnp.float32) + bcat_b_r[...]

    wu_blk = wu_r[pl.ds(k * KBLK, KBLK), :]
    wb_blk = wb_r[pl.ds(k * KBLK, KBLK), :]
    eu_s[...] += jnp.dot(users_r[...], wu_blk,
                         preferred_element_type=jnp.float32)
    eb_s[...] += jnp.dot(bus_r[...], wb_blk,
                         preferred_element_type=jnp.float32)
    nuu_s[...] += jnp.dot(uu_r[...], wu_blk,
                          preferred_element_type=jnp.float32)
    nub_s[...] += jnp.dot(ub_r[...], wb_blk,
                          preferred_element_type=jnp.float32)
    nbb_s[...] += jnp.dot(bb_r[...], wb_blk,
                          preferred_element_type=jnp.float32)
    nbu_s[...] += jnp.dot(bu_r[...], wu_blk,
                          preferred_element_type=jnp.float32)

    @pl.when(k == NK - 1)
    def _epilogue():
        S, St = _facet_mats()
        t_u = eu_s[...]
        t_b = eb_s[...]
        u1 = _homo(t_u, nuu_s[...], S, St)
        u2 = _homo(t_u, nub_s[...], S, St)
        u3 = _homo(t_u, nuc_s[...], S, St)
        u4 = _homo(t_u, nucat_s[...], S, St)
        hu = jnp.concatenate(
            [u1[:, None, :], u2[:, None, :], u3[:, None, :], u4[:, None, :]],
            axis=1).reshape(BB * F, FD)
        upd_u = _hete(t_u, hu, S, St)
        b1 = _homo(t_b, nbb_s[...], S, St)
        b2 = _homo(t_b, nbu_s[...], S, St)
        b3 = _homo(t_b, nbc_s[...], S, St)
        b4 = _homo(t_b, nbcat_s[...], S, St)
        hb = jnp.concatenate(
            [b1[:, None, :], b2[:, None, :], b3[:, None, :], b4[:, None, :]],
            axis=1).reshape(BB * F, FD)
        upd_b = _hete(t_u, hb, S, St)
        logit = jnp.sum(upd_u * upd_b, axis=1, keepdims=True)
        bidx = pl.program_id(0)
        out_r[pl.ds(bidx * BB, BB), :] = logit


def kernel(users, businesses, user_user_neigh, user_business_neigh,
           user_city_neigh, user_category_neigh, business_business_neigh,
           business_user_neigh, business_city_neigh, business_category_neigh,
           W_user, b_user, W_business, b_business, W_city, b_city,
           W_category, b_category):
    NBR = BK // NB

    grid = (NB, NK)
    big_spec = pl.BlockSpec((NBR, KBLK), lambda b, k: (b, k))
    tgt_spec = pl.BlockSpec((BB, KBLK), lambda b, k: (b, k))
    city_spec = pl.BlockSpec((NBR, NCITY), lambda b, k: (b, 0))
    cat_spec = pl.BlockSpec((NBR, NCAT), lambda b, k: (b, 0))
    wbig_spec = pl.BlockSpec((NBIG, FD), lambda b, k: (0, 0))
    wc_spec = pl.BlockSpec((NCITY, FD), lambda b, k: (0, 0))
    wcat_spec = pl.BlockSpec((NCAT, FD), lambda b, k: (0, 0))
    bias_spec = pl.BlockSpec((1, FD), lambda b, k: (0, 0))

    out = pl.pallas_call(
        _body,
        grid=grid,
        in_specs=[
            tgt_spec, tgt_spec,
            big_spec, big_spec,
            city_spec, cat_spec,
            big_spec, big_spec,
            city_spec, cat_spec,
            wbig_spec, bias_spec,
            wbig_spec, bias_spec,
            wc_spec, bias_spec,
            wcat_spec, bias_spec,
        ],
        out_specs=pl.BlockSpec((B, 1), lambda b, k: (0, 0)),
        out_shape=jax.ShapeDtypeStruct((B, 1), jnp.float32),
        scratch_shapes=[
            pltpu.VMEM((BB, FD), jnp.float32),
            pltpu.VMEM((BB, FD), jnp.float32),
            pltpu.VMEM((NBR, FD), jnp.float32),
            pltpu.VMEM((NBR, FD), jnp.float32),
            pltpu.VMEM((NBR, FD), jnp.float32),
            pltpu.VMEM((NBR, FD), jnp.float32),
            pltpu.VMEM((NBR, FD), jnp.float32),
            pltpu.VMEM((NBR, FD), jnp.float32),
            pltpu.VMEM((NBR, FD), jnp.float32),
            pltpu.VMEM((NBR, FD), jnp.float32),
        ],
        compiler_params=pltpu.CompilerParams(
            dimension_semantics=("arbitrary", "arbitrary"),
        ),
    )(users, businesses, user_user_neigh, user_business_neigh,
      user_city_neigh, user_category_neigh, business_business_neigh,
      business_user_neigh, business_city_neigh, business_category_neigh,
      W_user, b_user.reshape(1, FD), W_business, b_business.reshape(1, FD),
      W_city, b_city.reshape(1, FD), W_category, b_category.reshape(1, FD))
    return out.reshape(B)
